# Initial kernel scaffold; baseline (speedup 1.0000x reference)
import jax
import jax.numpy as jnp
from jax.experimental import pallas as pl


def kernel(x, emb, W, b):
    raise NotImplementedError("write your pallas kernel here")


# speedup vs baseline: 8.1906x; 8.1906x over previous
import functools

import jax
import jax.numpy as jnp
from jax import lax
from jax.experimental import pallas as pl
from jax.experimental.pallas import tpu as pltpu
from jax.experimental.pallas import tpu_sc as plsc

B = 4096
S = 200
H = 128
C = 1000

NC = 2
NS = 16
NW = NC * NS
LANES = 16
HCH = H // LANES

ROWS_PER_TILE = B // NW
CH = 4
N_CHUNK = ROWS_PER_TILE // CH
TOK = CH * S

_mesh = plsc.VectorSubcoreMesh(core_axis_name="c", subcore_axis_name="s")


@functools.partial(
    pl.kernel,
    mesh=_mesh,
    out_type=jax.ShapeDtypeStruct((B, H), jnp.float32),
    scratch_types=[
        pltpu.VMEM((TOK,), jnp.int32),
        pltpu.VMEM((TOK, H), jnp.float32),
        pltpu.VMEM((CH, H), jnp.float32),
        pltpu.SemaphoreType.DMA,
    ],
)
def _pool(x_hbm, emb_hbm, out_hbm, idx_v, rows_v, acc_v, sem):
    wid = lax.axis_index("s") * NC + lax.axis_index("c")
    row0 = wid * ROWS_PER_TILE

    def chunk_body(ci, carry):
        rbase = row0 + ci * CH
        tbase = rbase * S
        pltpu.sync_copy(x_hbm.at[pl.ds(tbase, TOK)], idx_v)
        pltpu.async_copy(emb_hbm.at[idx_v], rows_v, sem).wait()
        for r in range(CH):
            def s_body(si, accs):
                base = r * S + si
                return tuple(
                    a + rows_v[base, pl.ds(c * LANES, LANES)]
                    for c, a in enumerate(accs)
                )
            accs = lax.fori_loop(
                0, S, s_body,
                tuple(jnp.zeros((LANES,), jnp.float32) for _ in range(HCH)),
            )
            for c in range(HCH):
                acc_v[r, pl.ds(c * LANES, LANES)] = accs[c] * (1.0 / S)
        pltpu.sync_copy(acc_v, out_hbm.at[pl.ds(rbase, CH)])
        return carry

    lax.fori_loop(0, N_CHUNK, chunk_body, 0)


def _mm_body(bow_ref, w_ref, b_ref, out_ref):
    out_ref[...] = (
        lax.dot_general(
            bow_ref[...], w_ref[...],
            (((1,), (1,)), ((), ())),
            preferred_element_type=jnp.float32,
        )
        + b_ref[...]
    )


BLK = 512


def _matmul(bow, W, b2d):
    return pl.pallas_call(
        _mm_body,
        grid=(B // BLK,),
        in_specs=[
            pl.BlockSpec((BLK, H), lambda i: (i, 0)),
            pl.BlockSpec((C, H), lambda i: (0, 0)),
            pl.BlockSpec((1, C), lambda i: (0, 0)),
        ],
        out_specs=pl.BlockSpec((BLK, C), lambda i: (i, 0)),
        out_shape=jax.ShapeDtypeStruct((B, C), jnp.float32),
    )(bow, W, b2d)


def kernel(x, emb, W, b):
    bow = _pool(x.reshape(-1).astype(jnp.int32), emb)
    return _matmul(bow, W, b.reshape(1, C))


# speedup vs baseline: 13.3562x; 1.6307x over previous
import functools

import jax
import jax.numpy as jnp
from jax import lax
from jax.experimental import pallas as pl
from jax.experimental.pallas import tpu as pltpu
from jax.experimental.pallas import tpu_sc as plsc

B = 4096
S = 200
H = 128
C = 1000

NC = 2
NS = 16
NW = NC * NS
LANES = 16
HCH = H // LANES

ROWS_PER_TILE = B // NW
CH = 2
N_CHUNK = ROWS_PER_TILE // CH
TOK = CH * S

_mesh = plsc.VectorSubcoreMesh(core_axis_name="c", subcore_axis_name="s")


@functools.partial(
    pl.kernel,
    mesh=_mesh,
    out_type=jax.ShapeDtypeStruct((B, H), jnp.float32),
    scratch_types=[
        pltpu.VMEM((TOK,), jnp.int32),
        pltpu.VMEM((TOK,), jnp.int32),
        pltpu.VMEM((TOK, H), jnp.float32),
        pltpu.VMEM((TOK, H), jnp.float32),
        pltpu.VMEM((ROWS_PER_TILE, H), jnp.float32),
        pltpu.SemaphoreType.DMA,
        pltpu.SemaphoreType.DMA,
        pltpu.SemaphoreType.DMA,
        pltpu.SemaphoreType.DMA,
    ],
)
def _pool(x_hbm, emb_hbm, out_hbm,
          idx0, idx1, rows0, rows1, outst,
          sidx0, sidx1, srows0, srows1):
    idx = (idx0, idx1)
    rows = (rows0, rows1)
    sidx = (sidx0, sidx1)
    srows = (srows0, srows1)

    wid = lax.axis_index("s") * NC + lax.axis_index("c")
    row0 = wid * ROWS_PER_TILE
    tok0 = row0 * S

    def idx_copy(ci, p):
        return pltpu.make_async_copy(
            x_hbm.at[pl.ds(tok0 + ci * TOK, TOK)], idx[p], sidx[p])

    def gather(p):
        return pltpu.make_async_copy(emb_hbm.at[idx[p]], rows[p], srows[p])

    def reduce_compute(ci, p):
        rv = rows[p]

        def s_body(si, accs):
            s = si * 2
            new = []
            for r in range(CH):
                base = r * S
                for c in range(HCH):
                    new.append(
                        accs[r * HCH + c]
                        + rv[base + s, pl.ds(c * LANES, LANES)]
                        + rv[base + s + 1, pl.ds(c * LANES, LANES)]
                    )
            return tuple(new)

        accs = lax.fori_loop(
            0, S // 2, s_body,
            tuple(jnp.zeros((LANES,), jnp.float32) for _ in range(CH * HCH)),
        )
        for r in range(CH):
            lr = ci * CH + r
            for c in range(HCH):
                outst[lr, pl.ds(c * LANES, LANES)] = (
                    accs[r * HCH + c] * (1.0 / S))

    idx_copy(0, 0).start()
    idx_copy(1, 1).start()
    idx_copy(0, 0).wait()
    gather(0).start()

    def half_step(ci, p):
        @pl.when(ci + 1 < N_CHUNK)
        def _():
            idx_copy(ci + 1, 1 - p).wait()
            gather(1 - p).start()
        gather(p).wait()

        @pl.when(ci + 2 < N_CHUNK)
        def _():
            idx_copy(ci + 2, p).start()
        reduce_compute(ci, p)

    def body(pi, carry):
        half_step(pi * 2, 0)
        half_step(pi * 2 + 1, 1)
        return carry

    lax.fori_loop(0, N_CHUNK // 2, body, 0)
    pltpu.sync_copy(outst, out_hbm.at[pl.ds(row0, ROWS_PER_TILE)])


def _mm_body(bow_ref, w_ref, b_ref, out_ref):
    out_ref[...] = (
        lax.dot_general(
            bow_ref[...], w_ref[...],
            (((1,), (1,)), ((), ())),
            preferred_element_type=jnp.float32,
        )
        + b_ref[...]
    )


BLK = 512


def _matmul(bow, W, b2d):
    return pl.pallas_call(
        _mm_body,
        grid=(B // BLK,),
        in_specs=[
            pl.BlockSpec((BLK, H), lambda i: (i, 0)),
            pl.BlockSpec((C, H), lambda i: (0, 0)),
            pl.BlockSpec((1, C), lambda i: (0, 0)),
        ],
        out_specs=pl.BlockSpec((BLK, C), lambda i: (i, 0)),
        out_shape=jax.ShapeDtypeStruct((B, C), jnp.float32),
    )(bow, W, b2d)


def kernel(x, emb, W, b):
    bow = _pool(x.reshape(-1).astype(jnp.int32), emb)
    return _matmul(bow, W, b.reshape(1, C))
